# baseline (speedup 1.0000x reference)
import jax
import jax.numpy as jnp
from jax.experimental import pallas as pl
from jax.experimental.pallas import tpu as pltpu

TM2 = 512


def _softmax(logits):
    m = jnp.max(logits, axis=-1, keepdims=True)
    e = jnp.exp(logits - m)
    return e / jnp.sum(e, axis=-1, keepdims=True)


def _router_block(xe_ref, xo_ref, wt_ref, b_ref, out_ref):
    wt = wt_ref[...]
    bias = b_ref[...]
    pe = _softmax(jnp.dot(xe_ref[...].astype(jnp.bfloat16), wt,
                          preferred_element_type=jnp.float32) + bias)
    po = _softmax(jnp.dot(xo_ref[...].astype(jnp.bfloat16), wt,
                          preferred_element_type=jnp.float32) + bias)
    ne = pe.shape[-1]
    out_ref[:, :ne] = pe
    out_ref[:, ne:] = po


def kernel(x, W, b):
    tokens, d_model = x.shape
    num_experts = W.shape[0]
    wt = W.T.astype(jnp.bfloat16)
    b2 = b.reshape(1, num_experts)
    x2 = x.reshape(tokens // 2, 2 * d_model)
    grid = (tokens // 2 // TM2,)
    packed = pl.pallas_call(
        _router_block,
        grid=grid,
        in_specs=[
            pl.BlockSpec((TM2, d_model), lambda i: (i, 0)),
            pl.BlockSpec((TM2, d_model), lambda i: (i, 1)),
            pl.BlockSpec((d_model, num_experts), lambda i: (0, 0)),
            pl.BlockSpec((1, num_experts), lambda i: (0, 0)),
        ],
        out_specs=pl.BlockSpec((TM2, 2 * num_experts), lambda i: (i, 0)),
        out_shape=jax.ShapeDtypeStruct(
            (tokens // 2, 2 * num_experts), jnp.float32),
        compiler_params=pltpu.CompilerParams(
            dimension_semantics=("arbitrary",),
        ),
    )(x2, x2, wt, b2)
    return packed.reshape(tokens, num_experts)
